# Initial kernel scaffold; baseline (speedup 1.0000x reference)
import jax
import jax.numpy as jnp
from jax.experimental import pallas as pl


def kernel(input_ids, token_type_ids, W_word, W_tt, gamma, beta):
    raise NotImplementedError("write your pallas kernel here")


# speedup vs baseline: 1.6295x; 1.6295x over previous
import functools

import jax
import jax.numpy as jnp
from jax import lax
from jax.experimental import pallas as pl
from jax.experimental.pallas import tpu as pltpu
from jax.experimental.pallas import tpu_sc as plsc

VOCAB = 100000
MAXLEN = 2048
DIM = 768
B = 4
S = 2048
N = B * S

NC = 2
NS = 16
NW = NC * NS
ROWS_PER_W = N // NW
CHUNK = 64
NCHUNK = ROWS_PER_W // CHUNK


def _sc_gather_kernel(table_hbm, idx_hbm, out_hbm,
                      idx0, idx1, buf0, buf1, sem0, sem1):
  wid = lax.axis_index("s") * NC + lax.axis_index("c")
  base = pl.multiple_of(wid * ROWS_PER_W, ROWS_PER_W)

  idxs = (idx0, idx1)
  bufs = (buf0, buf1)
  sems = (sem0, sem1)

  def start(ci):
    off = pl.multiple_of(base + ci * CHUNK, CHUNK)
    slot = ci % 2
    pltpu.sync_copy(idx_hbm.at[pl.ds(off, CHUNK)], idxs[slot])
    return pltpu.async_copy(table_hbm.at[idxs[slot]], bufs[slot], sems[slot])

  cp = start(0)
  for ci in range(NCHUNK):
    nxt = start(ci + 1) if ci + 1 < NCHUNK else None
    cp.wait()
    off = pl.multiple_of(base + ci * CHUNK, CHUNK)
    pltpu.sync_copy(bufs[ci % 2], out_hbm.at[pl.ds(off, CHUNK)])
    cp = nxt


@jax.jit
def _sc_gather(table, idx):
  mesh = plsc.VectorSubcoreMesh(core_axis_name="c", subcore_axis_name="s")
  k = functools.partial(
      pl.kernel, mesh=mesh,
      out_type=jax.ShapeDtypeStruct((N, DIM), jnp.float32),
      scratch_types=[
          pltpu.VMEM((CHUNK,), jnp.int32),
          pltpu.VMEM((CHUNK,), jnp.int32),
          pltpu.VMEM((CHUNK, DIM), jnp.float32),
          pltpu.VMEM((CHUNK, DIM), jnp.float32),
          pltpu.SemaphoreType.DMA,
          pltpu.SemaphoreType.DMA,
      ],
  )(_sc_gather_kernel)
  return k(table, idx)


BS = 256
NB = N // BS
SB = S // BS


def _tc_ln_kernel(g_ref, pos_ref, tt_ref, w01_ref, gamma_ref, beta_ref,
                  out_ref):
  x = g_ref[...] + pos_ref[...]
  ttf = tt_ref[0, 0, :]
  row0 = w01_ref[0, :]
  drow = w01_ref[1, :] - row0
  x = x + row0[None, :] + ttf[:, None] * drow[None, :]
  mean = jnp.mean(x, axis=-1, keepdims=True)
  xc = x - mean
  var = jnp.mean(xc * xc, axis=-1, keepdims=True)
  y = xc * lax.rsqrt(var + 1e-5)
  out_ref[...] = y * gamma_ref[...] + beta_ref[...]


@jax.jit
def _tc_ln(gathered, W_tt, ttf, gamma2d, beta2d):
  return pl.pallas_call(
      _tc_ln_kernel,
      grid=(NB,),
      in_specs=[
          pl.BlockSpec((BS, DIM), lambda i: (i, 0)),
          pl.BlockSpec((BS, DIM), lambda i: (i % SB, 0)),
          pl.BlockSpec((1, 1, BS), lambda i: (i, 0, 0)),
          pl.BlockSpec((8, DIM), lambda i: (0, 0)),
          pl.BlockSpec((1, DIM), lambda i: (0, 0)),
          pl.BlockSpec((1, DIM), lambda i: (0, 0)),
      ],
      out_specs=pl.BlockSpec((BS, DIM), lambda i: (i, 0)),
      out_shape=jax.ShapeDtypeStruct((N, DIM), jnp.float32),
  )(gathered, W_tt, ttf, W_tt, gamma2d, beta2d)


def kernel(input_ids, token_type_ids, W_word, W_tt, gamma, beta):
  ids = input_ids.reshape(-1).astype(jnp.int32)
  gathered = _sc_gather(W_word, ids)
  ttf = token_type_ids.reshape(NB, 1, BS).astype(jnp.float32)
  out = _tc_ln(gathered, W_tt, ttf,
               gamma.reshape(1, DIM), beta.reshape(1, DIM))
  return out.reshape(B, S, DIM)


# speedup vs baseline: 1.8664x; 1.1454x over previous
import functools

import jax
import jax.numpy as jnp
from jax import lax
from jax.experimental import pallas as pl
from jax.experimental.pallas import tpu as pltpu
from jax.experimental.pallas import tpu_sc as plsc

VOCAB = 100000
MAXLEN = 2048
DIM = 768
B = 4
S = 2048
N = B * S

NC = 2
NS = 16
NW = NC * NS
ROWS_PER_W = N // NW
CHUNK = 64
NCHUNK = ROWS_PER_W // CHUNK


def _sc_gather_kernel(table_hbm, idx_hbm, out_hbm,
                      idx0, idx1, buf0, buf1, sem0, sem1):
  wid = lax.axis_index("s") * NC + lax.axis_index("c")
  base = pl.multiple_of(wid * ROWS_PER_W, ROWS_PER_W)

  idxs = (idx0, idx1)
  bufs = (buf0, buf1)
  sems = (sem0, sem1)

  def start(ci):
    off = pl.multiple_of(base + ci * CHUNK, CHUNK)
    slot = ci % 2
    pltpu.sync_copy(idx_hbm.at[pl.ds(off, CHUNK)], idxs[slot])
    return pltpu.async_copy(table_hbm.at[idxs[slot]], bufs[slot], sems[slot])

  cp = start(0)
  for ci in range(NCHUNK):
    nxt = start(ci + 1) if ci + 1 < NCHUNK else None
    cp.wait()
    off = pl.multiple_of(base + ci * CHUNK, CHUNK)
    pltpu.sync_copy(bufs[ci % 2], out_hbm.at[pl.ds(off, CHUNK)])
    cp = nxt


@jax.jit
def _sc_gather(table, idx):
  mesh = plsc.VectorSubcoreMesh(core_axis_name="c", subcore_axis_name="s")
  k = functools.partial(
      pl.kernel, mesh=mesh,
      out_type=jax.ShapeDtypeStruct((N, DIM), jnp.float32),
      scratch_types=[
          pltpu.VMEM((CHUNK,), jnp.int32),
          pltpu.VMEM((CHUNK,), jnp.int32),
          pltpu.VMEM((CHUNK, DIM), jnp.float32),
          pltpu.VMEM((CHUNK, DIM), jnp.float32),
          pltpu.SemaphoreType.DMA,
          pltpu.SemaphoreType.DMA,
      ],
  )(_sc_gather_kernel)
  return k(table, idx)


BS = 512
NB = N // BS
SB = S // BS


def _tc_ln_kernel(g_ref, pos_ref, tt_ref, w01_ref, gamma_ref, beta_ref,
                  out_ref):
  row0 = w01_ref[0, :]
  drow = w01_ref[1, :] - row0
  x = g_ref[...] + pos_ref[...]
  x = x + row0[None, :] + tt_ref[...] * drow[None, :]
  mean = jnp.mean(x, axis=-1, keepdims=True)
  xc = x - mean
  var = jnp.mean(xc * xc, axis=-1, keepdims=True)
  y = xc * lax.rsqrt(var + 1e-5)
  out_ref[...] = y * gamma_ref[...] + beta_ref[...]


@jax.jit
def _tc_ln(gathered, W_tt, ttf, gamma2d, beta2d):
  return pl.pallas_call(
      _tc_ln_kernel,
      grid=(SB, B),
      in_specs=[
          pl.BlockSpec((BS, DIM), lambda s, b: (b * SB + s, 0)),
          pl.BlockSpec((BS, DIM), lambda s, b: (s, 0)),
          pl.BlockSpec((BS, 1), lambda s, b: (b * SB + s, 0)),
          pl.BlockSpec((8, DIM), lambda s, b: (0, 0)),
          pl.BlockSpec((1, DIM), lambda s, b: (0, 0)),
          pl.BlockSpec((1, DIM), lambda s, b: (0, 0)),
      ],
      out_specs=pl.BlockSpec((BS, DIM), lambda s, b: (b * SB + s, 0)),
      out_shape=jax.ShapeDtypeStruct((N, DIM), jnp.float32),
  )(gathered, W_tt, ttf, W_tt, gamma2d, beta2d)


def kernel(input_ids, token_type_ids, W_word, W_tt, gamma, beta):
  ids = input_ids.reshape(-1).astype(jnp.int32)
  gathered = _sc_gather(W_word, ids)
  ttf = token_type_ids.reshape(N, 1).astype(jnp.float32)
  out = _tc_ln(gathered, W_tt, ttf,
               gamma.reshape(1, DIM), beta.reshape(1, DIM))
  return out.reshape(B, S, DIM)
